# baseline (speedup 1.0000x reference)
import functools

import jax
import jax.numpy as jnp
from jax import lax
from jax.experimental import pallas as pl
from jax.experimental.pallas import tpu as pltpu
from jax.experimental.pallas import tpu_sc as plsc

NC, NS, L = 2, 16, 16
NW = NC * NS
ROWS = 20000
PAD_ROWS = 20480
RPW = PAD_ROWS // NW
BLKS = RPW // L
GCH = 128
NG = RPW // GCH
H = W = 1024
HW = H * W

_mesh = plsc.VectorSubcoreMesh(core_axis_name="c", subcore_axis_name="s")


@functools.partial(
    pl.kernel,
    mesh=_mesh,
    out_type=jax.ShapeDtypeStruct((PAD_ROWS,), jnp.float32),
    scratch_types=[
        pltpu.VMEM((RPW * 7,), jnp.float32),
        pltpu.VMEM((RPW,), jnp.int32),
        pltpu.VMEM((RPW,), jnp.float32),
        pltpu.SemaphoreType.DMA,
    ],
    compiler_params=pltpu.CompilerParams(needs_layout_passes=False),
)
def _depth_gather(bflat_hbm, dm_hbm, out_hbm, bbuf, ibuf, dbuf, sem):
    wid = lax.axis_index("s") * NC + lax.axis_index("c")
    base = wid * RPW
    pltpu.sync_copy(bflat_hbm.at[pl.ds(base * 7, RPW * 7)], bbuf)
    lanes = lax.iota(jnp.int32, L)
    for r in range(BLKS):
        rows7 = (lanes + (r * L)) * 7
        bidf = plsc.load_gather(bbuf, [rows7])
        x1f = plsc.load_gather(bbuf, [rows7 + 3])
        y1f = plsc.load_gather(bbuf, [rows7 + 4])
        x2f = plsc.load_gather(bbuf, [rows7 + 5])
        y2f = plsc.load_gather(bbuf, [rows7 + 6])
        bid = jnp.clip(bidf.astype(jnp.int32), 0, 15)
        x1 = (x1f * W).astype(jnp.int32)
        y1 = (y1f * H).astype(jnp.int32)
        x2 = (x2f * W).astype(jnp.int32)
        y2 = (y2f * H).astype(jnp.int32)
        cx = jnp.clip(lax.shift_right_arithmetic(x1 + x2, 1), 0, W - 1)
        cy = jnp.clip(lax.shift_right_arithmetic(y1 + y2, 1), 0, H - 1)
        ibuf[pl.ds(r * L, L)] = bid * HW + cy * W + cx
    copies = [
        pltpu.async_copy(
            dm_hbm.at[ibuf.at[pl.ds(g * GCH, GCH)]],
            dbuf.at[pl.ds(g * GCH, GCH)],
            sem,
        )
        for g in range(NG)
    ]
    for cp in copies:
        cp.wait()
    pltpu.sync_copy(dbuf, out_hbm.at[pl.ds(base, RPW)])


def kernel(bboxes, depth_map):
    bflat = jnp.pad(bboxes, ((0, PAD_ROWS - ROWS), (0, 0))).reshape(-1)
    dm_flat = depth_map.reshape(-1)
    depths = _depth_gather(bflat, dm_flat)[:ROWS]
    return jnp.concatenate([bboxes, depths[:, None]], axis=1)


# speedup vs baseline: 1.1939x; 1.1939x over previous
import functools

import jax
import jax.numpy as jnp
from jax import lax
from jax.experimental import pallas as pl
from jax.experimental.pallas import tpu as pltpu
from jax.experimental.pallas import tpu_sc as plsc

NC, NS, L = 2, 16, 16
NW = NC * NS
ROWS = 20000
PAD_ROWS = 20480
RPW = PAD_ROWS // NW
BLKS = RPW // L
GCH = 128
NG = RPW // GCH
H = W = 1024
HW = H * W
TROWS = 16 * HW // 128

_mesh = plsc.VectorSubcoreMesh(core_axis_name="c", subcore_axis_name="s")


@functools.partial(
    pl.kernel,
    mesh=_mesh,
    out_type=jax.ShapeDtypeStruct((PAD_ROWS,), jnp.float32),
    scratch_types=[
        pltpu.VMEM((RPW * 7,), jnp.float32),
        pltpu.VMEM((RPW,), jnp.int32),
        pltpu.VMEM((RPW,), jnp.int32),
        pltpu.VMEM((RPW, 128), jnp.float32),
        pltpu.VMEM((RPW,), jnp.float32),
        pltpu.SemaphoreType.DMA,
    ],
    compiler_params=pltpu.CompilerParams(needs_layout_passes=False),
)
def _depth_gather(bflat_hbm, dmt_hbm, out_hbm, bbuf, ibuf, lbuf, gbuf, dbuf, sem):
    wid = lax.axis_index("s") * NC + lax.axis_index("c")
    base = wid * RPW
    pltpu.sync_copy(bflat_hbm.at[pl.ds(base * 7, RPW * 7)], bbuf)
    lanes = lax.iota(jnp.int32, L)
    for r in range(BLKS):
        rows7 = (lanes + (r * L)) * 7
        bidf = plsc.load_gather(bbuf, [rows7])
        x1f = plsc.load_gather(bbuf, [rows7 + 3])
        y1f = plsc.load_gather(bbuf, [rows7 + 4])
        x2f = plsc.load_gather(bbuf, [rows7 + 5])
        y2f = plsc.load_gather(bbuf, [rows7 + 6])
        bid = jnp.clip(bidf.astype(jnp.int32), 0, 15)
        x1 = (x1f * W).astype(jnp.int32)
        y1 = (y1f * H).astype(jnp.int32)
        x2 = (x2f * W).astype(jnp.int32)
        y2 = (y2f * H).astype(jnp.int32)
        cx = jnp.clip(lax.shift_right_arithmetic(x1 + x2, 1), 0, W - 1)
        cy = jnp.clip(lax.shift_right_arithmetic(y1 + y2, 1), 0, H - 1)
        trow = (
            bid * (HW // 128)
            + lax.shift_right_arithmetic(cy, 3) * 64
            + lax.shift_right_arithmetic(cx, 7) * 8
            + lax.bitwise_and(cy, 7)
        )
        ibuf[pl.ds(r * L, L)] = trow
        lbuf[pl.ds(r * L, L)] = lax.bitwise_and(cx, 127)
    copies = [
        pltpu.async_copy(
            dmt_hbm.at[ibuf.at[pl.ds(g * GCH, GCH)]],
            gbuf.at[pl.ds(g * GCH, GCH), :],
            sem,
        )
        for g in range(NG)
    ]
    for cp in copies:
        cp.wait()
    for r in range(BLKS):
        rows = lanes + (r * L)
        lane = lbuf[pl.ds(r * L, L)]
        dbuf[pl.ds(r * L, L)] = plsc.load_gather(gbuf, [rows, lane])
    pltpu.sync_copy(dbuf, out_hbm.at[pl.ds(base, RPW)])


def kernel(bboxes, depth_map):
    bflat = jnp.pad(bboxes, ((0, PAD_ROWS - ROWS), (0, 0))).reshape(-1)
    dmt = (
        depth_map.reshape(16, 128, 8, 8, 128)
        .transpose(0, 1, 3, 2, 4)
        .reshape(TROWS, 128)
    )
    depths = _depth_gather(bflat, dmt)[:ROWS]
    return jnp.concatenate([bboxes, depths[:, None]], axis=1)


# speedup vs baseline: 1.8121x; 1.5178x over previous
import functools

import jax
import jax.numpy as jnp
from jax import lax
from jax.experimental import pallas as pl
from jax.experimental.pallas import tpu as pltpu
from jax.experimental.pallas import tpu_sc as plsc

NC, NS, L = 2, 16, 16
NW = NC * NS
ROWS = 20000
PAD_ROWS = 20480
RPW = PAD_ROWS // NW
BLKS = RPW // L
GCH = 128
NG = RPW // GCH
H = W = 1024
HW = H * W
TROWS = 16 * HW // 128

_mesh = plsc.VectorSubcoreMesh(core_axis_name="c", subcore_axis_name="s")


@functools.partial(
    pl.kernel,
    mesh=_mesh,
    out_type=jax.ShapeDtypeStruct((PAD_ROWS,), jnp.float32),
    scratch_types=[
        pltpu.VMEM((RPW * 7,), jnp.float32),
        pltpu.VMEM((RPW,), jnp.int32),
        pltpu.VMEM((RPW,), jnp.float32),
        pltpu.SemaphoreType.DMA,
    ],
    compiler_params=pltpu.CompilerParams(needs_layout_passes=False),
)
def _depth_gather(bflat_hbm, dmt_hbm, out_hbm, bbuf, ibuf, dbuf, sem):
    wid = lax.axis_index("s") * NC + lax.axis_index("c")
    base = wid * RPW
    pltpu.sync_copy(bflat_hbm.at[pl.ds(base * 7, RPW * 7)], bbuf)
    lanes = lax.iota(jnp.int32, L)
    for r in range(BLKS):
        rows7 = (lanes + (r * L)) * 7
        bidf = plsc.load_gather(bbuf, [rows7])
        x1f = plsc.load_gather(bbuf, [rows7 + 3])
        y1f = plsc.load_gather(bbuf, [rows7 + 4])
        x2f = plsc.load_gather(bbuf, [rows7 + 5])
        y2f = plsc.load_gather(bbuf, [rows7 + 6])
        bid = jnp.clip(bidf.astype(jnp.int32), 0, 15)
        x1 = (x1f * W).astype(jnp.int32)
        y1 = (y1f * H).astype(jnp.int32)
        x2 = (x2f * W).astype(jnp.int32)
        y2 = (y2f * H).astype(jnp.int32)
        cx = jnp.clip(lax.shift_right_arithmetic(x1 + x2, 1), 0, W - 1)
        cy = jnp.clip(lax.shift_right_arithmetic(y1 + y2, 1), 0, H - 1)
        ibuf[pl.ds(r * L, L)] = (
            bid * HW
            + lax.shift_right_arithmetic(cy, 3) * 8192
            + lax.shift_right_arithmetic(cx, 7) * 1024
            + lax.bitwise_and(cy, 7) * 128
            + lax.bitwise_and(cx, 127)
        )
    copies = [
        pltpu.async_copy(
            dmt_hbm.at[ibuf.at[pl.ds(g * GCH, GCH)]],
            dbuf.at[pl.ds(g * GCH, GCH)],
            sem,
        )
        for g in range(NG)
    ]
    for cp in copies:
        cp.wait()
    pltpu.sync_copy(dbuf, out_hbm.at[pl.ds(base, RPW)])


def kernel(bboxes, depth_map):
    bflat = jnp.pad(bboxes, ((0, PAD_ROWS - ROWS), (0, 0))).reshape(-1)
    dmt = (
        depth_map.reshape(16, 128, 8, 8, 128)
        .transpose(0, 1, 3, 2, 4)
        .reshape(16 * HW)
    )
    depths = _depth_gather(bflat, dmt)[:ROWS]
    return jnp.concatenate([bboxes, depths[:, None]], axis=1)
